# Initial kernel scaffold; baseline (speedup 1.0000x reference)
import jax
import jax.numpy as jnp
from jax.experimental import pallas as pl


def kernel(x, fc_weight, bias, cb_index, codebooks):
    raise NotImplementedError("write your pallas kernel here")


# speedup vs baseline: 23.8955x; 23.8955x over previous
import dataclasses
import functools

import jax
import jax.numpy as jnp
from jax import lax
from jax.experimental import pallas as pl
from jax.experimental.pallas import tpu as pltpu
from jax.experimental.pallas import tpu_sc as plsc

F = 26
FIELD_SIZE = 40000
TOTAL = F * FIELD_SIZE
DIM = 128
M = 8
K = 256
PLEN = 16
B = 4096

NC = 2
NS = 16
NW = NC * NS

GROUPS = NW // M
B_FM = B // GROUPS
CHUNK = 128
NJ = CHUNK // 16
NCHUNKS = B_FM // CHUNK
B_LIN = B // NW


def _sc_partials(x_t, cb_flat, fc_flat, c_re):
  mesh = plsc.VectorSubcoreMesh(core_axis_name="c", subcore_axis_name="s")
  cp = pltpu.CompilerParams()
  if "needs_layout_passes" in pltpu.CompilerParams.__dataclass_fields__:
    cp = dataclasses.replace(cp, needs_layout_passes=False)

  @functools.partial(
      pl.kernel,
      compiler_params=cp,
      out_type=jax.ShapeDtypeStruct(((M + 1) * B,), jnp.float32),
      mesh=mesh,
      scratch_types=[
          pltpu.VMEM((F * K * PLEN,), jnp.float32),
          pltpu.VMEM((F, CHUNK), jnp.int32),
          pltpu.VMEM((F, CHUNK), jnp.int32),
          pltpu.VMEM((F, CHUNK), jnp.int32),
          pltpu.VMEM((F, CHUNK), jnp.float32),
          pltpu.VMEM((CHUNK,), jnp.float32),
          pltpu.SemaphoreType.DMA,
      ],
  )
  def sc_kernel(x_hbm, cb_hbm, fc_hbm, c_hbm, part_hbm,
                tab, xb, idxb, indb, fvb, outb, sem):
    wid = lax.axis_index("c") * NS + lax.axis_index("s")
    m = wid & (M - 1)
    g = wid >> 3

    tabn = F * K * PLEN
    pltpu.sync_copy(c_hbm.at[pl.ds(m * tabn, tabn)], tab)

    b0 = wid * B_LIN
    pltpu.sync_copy(x_hbm.at[:, pl.ds(b0, CHUNK)], xb)
    for f in range(F):
      for j in range(NJ):
        xv = xb[f, pl.ds(j * 16, 16)]
        idxb[f, pl.ds(j * 16, 16)] = xv + f * FIELD_SIZE
    descs = [pltpu.async_copy(fc_hbm.at[idxb.at[f]], fvb.at[f], sem)
             for f in range(F)]
    for d in descs:
      d.wait()
    for j in range(NJ):
      lin = fvb[0, pl.ds(j * 16, 16)]
      for f in range(1, F):
        lin = lin + fvb[f, pl.ds(j * 16, 16)]
      outb[pl.ds(j * 16, 16)] = lin
    pltpu.sync_copy(outb, part_hbm.at[pl.ds(M * B + b0, CHUNK)])

    @pl.loop(0, NCHUNKS)
    def _chunk(c):
      cb0 = g * B_FM + c * CHUNK
      pltpu.sync_copy(x_hbm.at[:, pl.ds(cb0, CHUNK)], xb)
      for f in range(F):
        coef = f * FIELD_SIZE * M + m
        for j in range(NJ):
          xv = xb[f, pl.ds(j * 16, 16)]
          idxb[f, pl.ds(j * 16, 16)] = (xv << 3) + coef
      ds2 = [pltpu.async_copy(cb_hbm.at[idxb.at[f]], indb.at[f], sem)
             for f in range(F)]
      for d in ds2:
        d.wait()

      @pl.loop(0, NJ)
      def _j(j):
        zero = jnp.zeros((16,), jnp.float32)
        acc = [zero] * PLEN
        sq = zero
        for f in range(F):
          ind = indb[f, pl.ds(j * 16, 16)]
          iv = (ind << 4) + f * K * PLEN
          for d in range(PLEN):
            v = plsc.load_gather(tab, [iv])
            iv = iv + 1
            acc[d] = acc[d] + v
            sq = sq + v * v
        fm = acc[0] * acc[0]
        for d in range(1, PLEN):
          fm = fm + acc[d] * acc[d]
        outb[pl.ds(j * 16, 16)] = 0.5 * (fm - sq)

      pltpu.sync_copy(outb, part_hbm.at[pl.ds(m * B + cb0, CHUNK)])

  return sc_kernel(x_t, cb_flat, fc_flat, c_re)


def _combine(parts, bias2d):
  def body(p_ref, b_ref, o_ref):
    o_ref[...] = jnp.sum(p_ref[...], axis=0, keepdims=True) + b_ref[...]

  return pl.pallas_call(
      body,
      out_shape=jax.ShapeDtypeStruct((1, B), jnp.float32),
  )(parts, bias2d)


@jax.jit
def kernel(x, fc_weight, bias, cb_index, codebooks):
  x_t = jnp.asarray(x, jnp.int32).T
  cb_flat = cb_index.reshape(TOTAL * M)
  fc_flat = fc_weight.reshape(TOTAL)
  c_re = codebooks.reshape(F * K, M, PLEN).transpose(1, 0, 2).reshape(-1)
  parts = _sc_partials(x_t, cb_flat, fc_flat, c_re)
  out = _combine(parts.reshape(M + 1, B), bias.reshape(1, 1).astype(jnp.float32))
  return out.reshape(B)


# speedup vs baseline: 24.0824x; 1.0078x over previous
import dataclasses
import functools

import jax
import jax.numpy as jnp
from jax import lax
from jax.experimental import pallas as pl
from jax.experimental.pallas import tpu as pltpu
from jax.experimental.pallas import tpu_sc as plsc

F = 26
FIELD_SIZE = 40000
TOTAL = F * FIELD_SIZE
DIM = 128
M = 8
K = 256
PLEN = 16
B = 4096

NC = 2
NS = 16
NW = NC * NS

GROUPS = NW // M
B_FM = B // GROUPS
CHUNK = 128
NJ = CHUNK // 16
NCHUNKS = B_FM // CHUNK
B_LIN = B // NW


def _sc_partials(x_t, cb_flat, fc_flat, c_re):
  mesh = plsc.VectorSubcoreMesh(core_axis_name="c", subcore_axis_name="s")
  cp = pltpu.CompilerParams()
  fields = pltpu.CompilerParams.__dataclass_fields__
  if "needs_layout_passes" in fields:
    cp = dataclasses.replace(cp, needs_layout_passes=False)
  if "use_tc_tiling_on_sc" in fields:
    cp = dataclasses.replace(cp, use_tc_tiling_on_sc=False)

  @functools.partial(
      pl.kernel,
      compiler_params=cp,
      out_type=jax.ShapeDtypeStruct(((M + 1) * B,), jnp.float32),
      mesh=mesh,
      scratch_types=[
          pltpu.VMEM((F * K, PLEN), jnp.float32),
          pltpu.VMEM((F, CHUNK), jnp.int32),
          pltpu.VMEM((F, CHUNK), jnp.int32),
          pltpu.VMEM((F, CHUNK), jnp.int32),
          pltpu.VMEM((F, CHUNK), jnp.float32),
          pltpu.VMEM((CHUNK,), jnp.float32),
          pltpu.SemaphoreType.DMA,
      ],
  )
  def sc_kernel(x_hbm, cb_hbm, fc_hbm, c_hbm, part_hbm,
                tab, xb, idxb, indb, fvb, outb, sem):
    wid = lax.axis_index("c") * NS + lax.axis_index("s")
    m = wid & (M - 1)
    g = wid >> 3

    pltpu.sync_copy(c_hbm.at[:, pl.ds(m * PLEN, PLEN)], tab)

    b0 = wid * B_LIN
    pltpu.sync_copy(x_hbm.at[:, pl.ds(b0, CHUNK)], xb)
    for f in range(F):
      for j in range(NJ):
        xv = xb[f, pl.ds(j * 16, 16)]
        idxb[f, pl.ds(j * 16, 16)] = xv + f * FIELD_SIZE
    descs = [pltpu.async_copy(fc_hbm.at[idxb.at[f]], fvb.at[f], sem)
             for f in range(F)]
    for d in descs:
      d.wait()
    for j in range(NJ):
      lin = fvb[0, pl.ds(j * 16, 16)]
      for f in range(1, F):
        lin = lin + fvb[f, pl.ds(j * 16, 16)]
      outb[pl.ds(j * 16, 16)] = lin
    pltpu.sync_copy(outb, part_hbm.at[pl.ds(M * B + b0, CHUNK)])

    @pl.loop(0, NCHUNKS)
    def _chunk(c):
      cb0 = g * B_FM + c * CHUNK
      pltpu.sync_copy(x_hbm.at[:, pl.ds(cb0, CHUNK)], xb)
      for f in range(F):
        coef = f * FIELD_SIZE * M + m
        for j in range(NJ):
          xv = xb[f, pl.ds(j * 16, 16)]
          idxb[f, pl.ds(j * 16, 16)] = (xv << 3) + coef
      ds2 = [pltpu.async_copy(cb_hbm.at[idxb.at[f]], indb.at[f], sem)
             for f in range(F)]
      for d in ds2:
        d.wait()

      cols = [jnp.full((16,), d, jnp.int32) for d in range(PLEN)]

      @pl.loop(0, NJ)
      def _j(j):
        zero = jnp.zeros((16,), jnp.float32)
        acc = [zero] * PLEN
        sq = zero
        for f in range(F):
          ind = indb[f, pl.ds(j * 16, 16)]
          row = ind + f * K
          for d in range(PLEN):
            v = plsc.load_gather(tab, [row, cols[d]])
            acc[d] = acc[d] + v
            sq = sq + v * v
        fm = acc[0] * acc[0]
        for d in range(1, PLEN):
          fm = fm + acc[d] * acc[d]
        outb[pl.ds(j * 16, 16)] = 0.5 * (fm - sq)

      pltpu.sync_copy(outb, part_hbm.at[pl.ds(m * B + cb0, CHUNK)])

  return sc_kernel(x_t, cb_flat, fc_flat, c_re)


def _combine(parts, bias2d):
  def body(p_ref, b_ref, o_ref):
    o_ref[...] = jnp.sum(p_ref[...], axis=0, keepdims=True) + b_ref[...]

  return pl.pallas_call(
      body,
      out_shape=jax.ShapeDtypeStruct((1, B), jnp.float32),
  )(parts, bias2d)


@jax.jit
def kernel(x, fc_weight, bias, cb_index, codebooks):
  x_t = jnp.asarray(x, jnp.int32).T
  cb_flat = cb_index.reshape(TOTAL * M)
  fc_flat = fc_weight.reshape(TOTAL)
  parts = _sc_partials(x_t, cb_flat, fc_flat, codebooks)
  out = _combine(parts.reshape(M + 1, B), bias.reshape(1, 1).astype(jnp.float32))
  return out.reshape(B)


# speedup vs baseline: 72.5536x; 3.0127x over previous
import dataclasses
import functools

import jax
import jax.numpy as jnp
from jax import lax
from jax.experimental import pallas as pl
from jax.experimental.pallas import tpu as pltpu
from jax.experimental.pallas import tpu_sc as plsc

F = 26
FIELD_SIZE = 40000
TOTAL = F * FIELD_SIZE
DIM = 128
M = 8
K = 256
PLEN = 16
B = 4096

NC = 2
NS = 16
NW = NC * NS

GROUPS = NW // M
B_FM = B // GROUPS
CHUNK = 128
NJ = CHUNK // 16
NCHUNKS = B_FM // CHUNK
B_LIN = B // NW
CHUNK_BYTES = F * CHUNK * 4


def _sc_partials(x_t, cb_bits, fc_flat, cbk):
  mesh = plsc.VectorSubcoreMesh(core_axis_name="c", subcore_axis_name="s")
  cp = pltpu.CompilerParams()
  fields = pltpu.CompilerParams.__dataclass_fields__
  if "needs_layout_passes" in fields:
    cp = dataclasses.replace(cp, needs_layout_passes=False)
  if "use_tc_tiling_on_sc" in fields:
    cp = dataclasses.replace(cp, use_tc_tiling_on_sc=False)

  @functools.partial(
      pl.kernel,
      compiler_params=cp,
      out_type=jax.ShapeDtypeStruct(((M + 1) * B,), jnp.float32),
      mesh=mesh,
      scratch_types=[
          pltpu.VMEM((F * K, PLEN), jnp.float32),
          pltpu.VMEM((2, F, CHUNK), jnp.int32),
          pltpu.VMEM((2, F, CHUNK), jnp.float32),
          pltpu.VMEM((CHUNK,), jnp.float32),
          pltpu.SemaphoreType.DMA((2,)),
      ],
  )
  def sc_kernel(x_hbm, cb_hbm, fc_hbm, c_hbm, part_hbm,
                tab, ib, vb, outb, semd):
    wid = lax.axis_index("c") * NS + lax.axis_index("s")
    m = wid & (M - 1)
    g = wid >> 3
    mbase = m << 7

    pltpu.sync_copy(c_hbm.at[:, pl.ds(m * PLEN, PLEN)], tab)

    cols = [jnp.full((16,), d, jnp.int32) for d in range(PLEN)]

    def drain(buf):
      pltpu.make_async_copy(
          x_hbm.at[:, pl.ds(0, CHUNK)], ib.at[buf], semd.at[buf]).wait()

    def build_idx(buf, cb0):
      pltpu.sync_copy(x_hbm.at[:, pl.ds(cb0, CHUNK)], ib.at[buf])
      for f in range(F):
        coef = f * FIELD_SIZE
        for j in range(NJ):
          xv = ib[buf, f, pl.ds(j * 16, 16)] + coef
          ib[buf, f, pl.ds(j * 16, 16)] = ((xv >> 7) << 10) + (xv & 127) + mbase

    def fire(buf):
      for f in range(F):
        pltpu.async_copy(cb_hbm.at[ib.at[buf, f]], vb.at[buf, f], semd.at[buf])

    b0 = wid * B_LIN
    pltpu.sync_copy(x_hbm.at[:, pl.ds(b0, CHUNK)], ib.at[0])
    for f in range(F):
      for j in range(NJ):
        xv = ib[0, f, pl.ds(j * 16, 16)]
        ib[0, f, pl.ds(j * 16, 16)] = xv + f * FIELD_SIZE
    for f in range(F):
      pltpu.async_copy(fc_hbm.at[ib.at[0, f]], vb.at[0, f], semd.at[0])
    drain(0)
    for j in range(NJ):
      lin = vb[0, 0, pl.ds(j * 16, 16)]
      for f in range(1, F):
        lin = lin + vb[0, f, pl.ds(j * 16, 16)]
      outb[pl.ds(j * 16, 16)] = lin
    pltpu.sync_copy(outb, part_hbm.at[pl.ds(M * B + b0, CHUNK)])

    build_idx(0, g * B_FM)
    fire(0)

    @pl.loop(0, NCHUNKS)
    def _chunk(c):
      buf = c & 1
      nbuf = 1 - buf
      drain(buf)

      @pl.when(c < NCHUNKS - 1)
      def _():
        build_idx(nbuf, g * B_FM + (c + 1) * CHUNK)
        fire(nbuf)

      @pl.loop(0, NJ)
      def _j(j):
        zero = jnp.zeros((16,), jnp.float32)
        acc = [zero] * PLEN
        sq = zero
        for f in range(F):
          ind = plsc.bitcast(vb[buf, f, pl.ds(j * 16, 16)], jnp.int32)
          row = ind + f * K
          for d in range(PLEN):
            v = plsc.load_gather(tab, [row, cols[d]])
            acc[d] = acc[d] + v
            sq = sq + v * v
        fm = acc[0] * acc[0]
        for d in range(1, PLEN):
          fm = fm + acc[d] * acc[d]
        outb[pl.ds(j * 16, 16)] = 0.5 * (fm - sq)

      pltpu.sync_copy(
          outb, part_hbm.at[pl.ds(m * B + g * B_FM + c * CHUNK, CHUNK)])

  return sc_kernel(x_t, cb_bits, fc_flat, cbk)


def _combine(parts, bias2d):
  def body(p_ref, b_ref, o_ref):
    o_ref[...] = jnp.sum(p_ref[...], axis=0, keepdims=True) + b_ref[...]

  return pl.pallas_call(
      body,
      out_shape=jax.ShapeDtypeStruct((1, B), jnp.float32),
  )(parts, bias2d)


@jax.jit
def kernel(x, fc_weight, bias, cb_index, codebooks):
  x_t = jnp.asarray(x, jnp.int32).T
  cb_flat = (cb_index.T.reshape(M, TOTAL // 128, 128)
             .transpose(1, 0, 2).reshape(M * TOTAL))
  cb_bits = lax.bitcast_convert_type(cb_flat, jnp.float32)
  fc_flat = fc_weight.reshape(TOTAL)
  parts = _sc_partials(x_t, cb_bits, fc_flat, codebooks)
  out = _combine(parts.reshape(M + 1, B), bias.reshape(1, 1).astype(jnp.float32))
  return out.reshape(B)


# speedup vs baseline: 80.1779x; 1.1051x over previous
import dataclasses
import functools

import jax
import jax.numpy as jnp
from jax import lax
from jax.experimental import pallas as pl
from jax.experimental.pallas import tpu as pltpu
from jax.experimental.pallas import tpu_sc as plsc

F = 26
FIELD_SIZE = 40000
TOTAL = F * FIELD_SIZE
DIM = 128
M = 8
K = 256
PLEN = 16
B = 4096

NC = 2
NS = 16
NW = NC * NS

GROUPS = NW // M
B_FM = B // GROUPS
CHUNK = 128
NJ = CHUNK // 16
NCHUNKS = B_FM // CHUNK
B_LIN = B // NW
CHUNK_BYTES = F * CHUNK * 4


def _sc_partials(x_t, cb_bits, fc_flat, cbk):
  mesh = plsc.VectorSubcoreMesh(core_axis_name="c", subcore_axis_name="s")
  cp = pltpu.CompilerParams()
  fields = pltpu.CompilerParams.__dataclass_fields__
  if "needs_layout_passes" in fields:
    cp = dataclasses.replace(cp, needs_layout_passes=False)
  if "use_tc_tiling_on_sc" in fields:
    cp = dataclasses.replace(cp, use_tc_tiling_on_sc=False)

  @functools.partial(
      pl.kernel,
      compiler_params=cp,
      out_type=jax.ShapeDtypeStruct(((M + 1) * B,), jnp.float32),
      mesh=mesh,
      scratch_types=[
          pltpu.VMEM((F * K, PLEN), jnp.float32),
          pltpu.VMEM((2, F, CHUNK), jnp.int32),
          pltpu.VMEM((2, F, CHUNK), jnp.float32),
          pltpu.VMEM((CHUNK,), jnp.float32),
          pltpu.SemaphoreType.DMA((2,)),
      ],
  )
  def sc_kernel(x_hbm, cb_hbm, fc_hbm, c_hbm, part_hbm,
                tab, ib, vb, outb, semd):
    wid = lax.axis_index("c") * NS + lax.axis_index("s")
    m = wid & (M - 1)
    g = wid >> 3
    mbase = m << 7

    pltpu.sync_copy(c_hbm.at[:, pl.ds(m * PLEN, PLEN)], tab)

    cols = [jnp.full((16,), d, jnp.int32) for d in range(PLEN)]

    def drain(buf):
      pltpu.make_async_copy(
          x_hbm.at[:, pl.ds(0, CHUNK)], ib.at[buf], semd.at[buf]).wait()

    def build_idx(buf, cb0):
      pltpu.sync_copy(x_hbm.at[:, pl.ds(cb0, CHUNK)], ib.at[buf])
      for f in range(F):
        coef = f * FIELD_SIZE
        for j in range(NJ):
          xv = ib[buf, f, pl.ds(j * 16, 16)] + coef
          ib[buf, f, pl.ds(j * 16, 16)] = ((xv >> 7) << 10) + (xv & 127) + mbase

    def fire(buf):
      for f in range(F):
        pltpu.async_copy(cb_hbm.at[ib.at[buf, f]], vb.at[buf, f], semd.at[buf])

    b0 = wid * B_LIN
    pltpu.sync_copy(x_hbm.at[:, pl.ds(b0, CHUNK)], ib.at[0])
    for f in range(F):
      for j in range(NJ):
        xv = ib[0, f, pl.ds(j * 16, 16)]
        ib[0, f, pl.ds(j * 16, 16)] = xv + f * FIELD_SIZE
    for f in range(F):
      pltpu.async_copy(fc_hbm.at[ib.at[0, f]], vb.at[0, f], semd.at[0])
    drain(0)
    for j in range(NJ):
      lin = vb[0, 0, pl.ds(j * 16, 16)]
      for f in range(1, F):
        lin = lin + vb[0, f, pl.ds(j * 16, 16)]
      outb[pl.ds(j * 16, 16)] = lin
    pltpu.sync_copy(outb, part_hbm.at[pl.ds(M * B + b0, CHUNK)])

    build_idx(0, g * B_FM)
    fire(0)

    @pl.loop(0, NCHUNKS)
    def _chunk(c):
      buf = c & 1
      nbuf = 1 - buf
      drain(buf)

      @pl.when(c < NCHUNKS - 1)
      def _():
        build_idx(nbuf, g * B_FM + (c + 1) * CHUNK)
        fire(nbuf)

      @pl.loop(0, NJ)
      def _j(j):
        zero = jnp.zeros((16,), jnp.float32)
        acc = [zero] * PLEN
        sqa = [zero] * PLEN
        for f in range(F):
          ind = plsc.bitcast(vb[buf, f, pl.ds(j * 16, 16)], jnp.int32)
          row = ind + f * K
          for d in range(PLEN):
            v = plsc.load_gather(tab, [row, cols[d]])
            acc[d] = acc[d] + v
            sqa[d] = sqa[d] + v * v
        fm = acc[0] * acc[0] - sqa[0]
        for d in range(1, PLEN):
          fm = fm + (acc[d] * acc[d] - sqa[d])
        outb[pl.ds(j * 16, 16)] = 0.5 * fm

      pltpu.sync_copy(
          outb, part_hbm.at[pl.ds(m * B + g * B_FM + c * CHUNK, CHUNK)])

  return sc_kernel(x_t, cb_bits, fc_flat, cbk)


def _combine(parts, bias2d):
  def body(p_ref, b_ref, o_ref):
    o_ref[...] = jnp.sum(p_ref[...], axis=0, keepdims=True) + b_ref[...]

  return pl.pallas_call(
      body,
      out_shape=jax.ShapeDtypeStruct((1, B), jnp.float32),
  )(parts, bias2d)


@jax.jit
def kernel(x, fc_weight, bias, cb_index, codebooks):
  x_t = jnp.asarray(x, jnp.int32).T
  cb_flat = (cb_index.T.reshape(M, TOTAL // 128, 128)
             .transpose(1, 0, 2).reshape(M * TOTAL))
  cb_bits = lax.bitcast_convert_type(cb_flat, jnp.float32)
  fc_flat = fc_weight.reshape(TOTAL)
  parts = _sc_partials(x_t, cb_bits, fc_flat, codebooks)
  out = _combine(parts.reshape(M + 1, B), bias.reshape(1, 1).astype(jnp.float32))
  return out.reshape(B)
